# baseline (speedup 1.0000x reference)
import dataclasses
import functools

import jax
import jax.numpy as jnp
from jax import lax
from jax.experimental import pallas as pl
from jax.experimental.pallas import tpu as pltpu
from jax.experimental.pallas import tpu_sc as plsc

N = 10000
E = 160000
IN_C = 256
C = 512
NPAD = 10240
NSUB = 16
NCORE = 2
NPASS = 2
LANES = 16
EPT = E // NSUB
ROWS_PER_TILE = NPAD // (NSUB * NCORE * NPASS)
KB = 64
SEG = 2000
CBUF = SEG + KB + LANES
RB = 1000
GRID = N // RB

_mesh = plsc.VectorSubcoreMesh(core_axis_name="c", subcore_axis_name="s")

_sc_params = pltpu.CompilerParams()
if "needs_layout_passes" in pltpu.CompilerParams.__dataclass_fields__:
    _sc_params = dataclasses.replace(_sc_params, needs_layout_passes=False)


def _vec_loop(total):
    return range(total // LANES)


def _rsqrt_newton(x):
    i = plsc.bitcast(x, jnp.int32)
    i = jnp.int32(0x5F3759DF) - lax.shift_right_logical(i, 1)
    y = plsc.bitcast(i, jnp.float32)
    for _ in range(3):
        y = y * (1.5 - 0.5 * x * y * y)
    return y


def _sc_degnorm_body(dst_hbm, ew_hbm, dinv_hbm, dst_v, w_v, deg_v, col_v, acc_v,
                     stage_sh):
    cid = lax.axis_index("c")
    sid = lax.axis_index("s")

    @pl.when(cid == 0)
    def _():
        base = sid * EPT
        pltpu.sync_copy(dst_hbm.at[pl.ds(base, EPT)], dst_v)
        pltpu.sync_copy(ew_hbm.at[pl.ds(base, EPT)], w_v)
        zero = jnp.zeros((LANES,), jnp.float32)

        @pl.loop(0, NPAD // LANES)
        def _(j):
            deg_v[pl.ds(j * LANES, LANES)] = zero

        @pl.loop(0, EPT // LANES)
        def _(i):
            dvec = dst_v[pl.ds(i * LANES, LANES)]
            wvec = w_v[pl.ds(i * LANES, LANES)]
            plsc.addupdate_scatter(deg_v, [dvec], wvec)

        pltpu.sync_copy(deg_v, stage_sh.at[sid])
        plsc.subcore_barrier()

        width = NPAD // NSUB
        one = jnp.full((LANES,), 1.0, jnp.float32)
        for j in _vec_loop(width):
            acc_v[pl.ds(j * LANES, LANES)] = one
        for p in range(NSUB):
            pltpu.sync_copy(stage_sh.at[p].at[pl.ds(sid * width, width)], col_v)
            for j in _vec_loop(width):
                sl = pl.ds(j * LANES, LANES)
                acc_v[sl] = acc_v[sl] + col_v[sl]
        for j in _vec_loop(width):
            sl = pl.ds(j * LANES, LANES)
            acc_v[sl] = _rsqrt_newton(acc_v[sl])
        pltpu.sync_copy(acc_v, dinv_hbm.at[pl.ds(sid * width, width)])


def _sc_degnorm(dst, ew):
    k = pl.kernel(
        _sc_degnorm_body,
        out_type=jax.ShapeDtypeStruct((NPAD,), jnp.float32),
        mesh=_mesh,
        compiler_params=_sc_params,
        scratch_types=[
            pltpu.VMEM((EPT,), jnp.int32),
            pltpu.VMEM((EPT,), jnp.float32),
            pltpu.VMEM((NPAD,), jnp.float32),
            pltpu.VMEM((NPAD // NSUB,), jnp.float32),
            pltpu.VMEM((NPAD // NSUB,), jnp.float32),
            pltpu.VMEM_SHARED((NSUB, NPAD), jnp.float32),
        ],
    )
    return k(dst, ew)


def _sc_agg_body(hp_hbm, src_hbm, dst_hbm, ew_hbm, z_hbm, sseg_v, dseg_v,
                 wseg_v, csrc_v, cdst_v, cw_v, gbuf_v, acc_v):
    cid = lax.axis_index("c")
    sid = lax.axis_index("s")
    tid = cid * NSUB + sid
    zvec = jnp.zeros((LANES,), jnp.float32)
    zi = jnp.zeros((LANES,), jnp.int32)

    for p in range(NPASS):
        g = p * (NSUB * NCORE) + tid
        r0 = g * ROWS_PER_TILE

        @pl.loop(0, ROWS_PER_TILE)
        def _(r):
            for j in _vec_loop(C):
                acc_v[r, pl.ds(j * LANES, LANES)] = zvec

        @pl.loop(0, E // SEG)
        def _(seg):
            soff = seg * SEG
            pltpu.sync_copy(src_hbm.at[pl.ds(soff, SEG)], sseg_v)
            pltpu.sync_copy(dst_hbm.at[pl.ds(soff, SEG)], dseg_v)
            pltpu.sync_copy(ew_hbm.at[pl.ds(soff, SEG)], wseg_v)

            def fbody(i, nh):
                sl = pl.ds(i * LANES, LANES)
                dvec = dseg_v[sl]
                m = (dvec >= r0) & (dvec < r0 + ROWS_PER_TILE)
                plsc.store_compressed(csrc_v.at[pl.ds(nh, LANES)], sseg_v[sl],
                                      mask=m)
                plsc.store_compressed(cdst_v.at[pl.ds(nh, LANES)], dvec - r0,
                                      mask=m)
                plsc.store_compressed(cw_v.at[pl.ds(nh, LANES)], wseg_v[sl],
                                      mask=m)
                return nh + lax.reduce_sum(m.astype(jnp.int32), axes=(0,))

            nh = lax.fori_loop(0, SEG // LANES, fbody, jnp.int32(0))

            for j in range(KB // LANES):
                sl = pl.ds(nh + j * LANES, LANES)
                csrc_v[sl] = zi
                cdst_v[sl] = zi
                cw_v[sl] = zvec

            nb = (nh + (KB - 1)) // KB

            def pbody(b, _):
                off = b * KB
                pltpu.sync_copy(hp_hbm.at[csrc_v.at[pl.ds(off, KB)]], gbuf_v)

                @pl.loop(0, KB // LANES)
                def _(kv):
                    dv = cdst_v[pl.ds(off + kv * LANES, LANES)]
                    wv = cw_v[pl.ds(off + kv * LANES, LANES)]
                    for k2 in range(LANES):
                        d = dv[k2]
                        wb = lax.broadcast(wv[k2], (LANES,))
                        k = kv * LANES + k2
                        for j in _vec_loop(C):
                            sl = pl.ds(j * LANES, LANES)
                            acc_v[d, sl] = acc_v[d, sl] + wb * gbuf_v[k, sl]
                return 0

            lax.fori_loop(0, nb, pbody, 0)

        pltpu.sync_copy(acc_v, z_hbm.at[pl.ds(r0, ROWS_PER_TILE)])


def _sc_agg(hp, src, dst, ew):
    k = pl.kernel(
        _sc_agg_body,
        out_type=jax.ShapeDtypeStruct((NPAD, C), jnp.float32),
        mesh=_mesh,
        compiler_params=_sc_params,
        scratch_types=[
            pltpu.VMEM((SEG,), jnp.int32),
            pltpu.VMEM((SEG,), jnp.int32),
            pltpu.VMEM((SEG,), jnp.float32),
            pltpu.VMEM((CBUF,), jnp.int32),
            pltpu.VMEM((CBUF,), jnp.int32),
            pltpu.VMEM((CBUF,), jnp.float32),
            pltpu.VMEM((KB, C), jnp.float32),
            pltpu.VMEM((ROWS_PER_TILE, C), jnp.float32),
        ],
    )
    return k(hp, src, dst, ew)


def _mm_scale_body(x_ref, w_ref, dinv_ref, o_ref):
    o_ref[...] = dinv_ref[...] * jnp.dot(
        x_ref[...], w_ref[...], preferred_element_type=jnp.float32)


def _tc_linear_scale(x, W, dinv2d):
    return pl.pallas_call(
        _mm_scale_body,
        grid=(GRID,),
        in_specs=[
            pl.BlockSpec((RB, x.shape[1]), lambda i: (i, 0)),
            pl.BlockSpec(W.shape, lambda i: (0, 0)),
            pl.BlockSpec((RB, 1), lambda i: (i, 0)),
        ],
        out_specs=pl.BlockSpec((RB, C), lambda i: (i, 0)),
        out_shape=jax.ShapeDtypeStruct((N, C), jnp.float32),
    )(x, W, dinv2d)


def _stats_body(z_ref, hp_ref, dinv_ref, b_ref, t_ref, s_ref):
    t = dinv_ref[...] * (z_ref[...] + hp_ref[...]) + b_ref[...]
    t_ref[...] = t
    part = jnp.concatenate(
        [jnp.sum(t, axis=0, keepdims=True),
         jnp.sum(t * t, axis=0, keepdims=True)], axis=0)

    @pl.when(pl.program_id(0) == 0)
    def _():
        s_ref[...] = part

    @pl.when(pl.program_id(0) != 0)
    def _():
        s_ref[...] = s_ref[...] + part


def _tc_stats(z, hp, dinv2d, b):
    return pl.pallas_call(
        _stats_body,
        grid=(GRID,),
        in_specs=[
            pl.BlockSpec((RB, C), lambda i: (i, 0)),
            pl.BlockSpec((RB, C), lambda i: (i, 0)),
            pl.BlockSpec((RB, 1), lambda i: (i, 0)),
            pl.BlockSpec((1, C), lambda i: (0, 0)),
        ],
        out_specs=[
            pl.BlockSpec((RB, C), lambda i: (i, 0)),
            pl.BlockSpec((2, C), lambda i: (0, 0)),
        ],
        out_shape=[
            jax.ShapeDtypeStruct((N, C), jnp.float32),
            jax.ShapeDtypeStruct((2, C), jnp.float32),
        ],
    )(z, hp, dinv2d, b)


def _bn_from_sums(t, s, g, be):
    m = s[0:1, :] * (1.0 / N)
    v = s[1:2, :] * (1.0 / N) - m * m
    return jnp.maximum((t - m) * lax.rsqrt(v + 1e-5) * g + be, 0.0)


def _bnmm_body(t_ref, s_ref, g_ref, be_ref, w_ref, dinv_ref, o_ref):
    u = _bn_from_sums(t_ref[...], s_ref[...], g_ref[...], be_ref[...])
    o_ref[...] = dinv_ref[...] * jnp.dot(
        u, w_ref[...], preferred_element_type=jnp.float32)


def _tc_bn_relu_linear_scale(t, s, g, be, W, dinv2d):
    return pl.pallas_call(
        _bnmm_body,
        grid=(GRID,),
        in_specs=[
            pl.BlockSpec((RB, C), lambda i: (i, 0)),
            pl.BlockSpec((2, C), lambda i: (0, 0)),
            pl.BlockSpec((1, C), lambda i: (0, 0)),
            pl.BlockSpec((1, C), lambda i: (0, 0)),
            pl.BlockSpec((C, C), lambda i: (0, 0)),
            pl.BlockSpec((RB, 1), lambda i: (i, 0)),
        ],
        out_specs=pl.BlockSpec((RB, C), lambda i: (i, 0)),
        out_shape=jax.ShapeDtypeStruct((N, C), jnp.float32),
    )(t, s, g, be, W, dinv2d)


def _final_body(t_ref, s_ref, g_ref, be_ref, dist_ref, degf_ref, wd_ref,
                bd_ref, wdeg_ref, bdeg_ref, wmh_ref, wmd_ref, wmg_ref, bm_ref,
                o_ref):
    u = _bn_from_sums(t_ref[...], s_ref[...], g_ref[...], be_ref[...])
    dfe = jnp.maximum(dist_ref[...] * wd_ref[...] + bd_ref[...], 0.0)
    gfe = jnp.maximum(degf_ref[...] * wdeg_ref[...] + bdeg_ref[...], 0.0)
    acc = jnp.dot(u, wmh_ref[...], preferred_element_type=jnp.float32)
    acc = acc + jnp.dot(dfe, wmd_ref[...], preferred_element_type=jnp.float32)
    acc = acc + jnp.dot(gfe, wmg_ref[...], preferred_element_type=jnp.float32)
    o_ref[...] = acc + bm_ref[...]


def _tc_final(t, s, g, be, dist2d, degf2d, Wd, bd, Wdeg, bdeg, Wmh, Wmd, Wmg,
              bm):
    col = pl.BlockSpec((RB, 1), lambda i: (i, 0))
    rowv = pl.BlockSpec((1, C), lambda i: (0, 0))
    mat = pl.BlockSpec((C, C), lambda i: (0, 0))
    return pl.pallas_call(
        _final_body,
        grid=(GRID,),
        in_specs=[
            pl.BlockSpec((RB, C), lambda i: (i, 0)),
            pl.BlockSpec((2, C), lambda i: (0, 0)),
            rowv, rowv, col, col, rowv, rowv, rowv, rowv, mat, mat, mat, rowv,
        ],
        out_specs=pl.BlockSpec((RB, C), lambda i: (i, 0)),
        out_shape=jax.ShapeDtypeStruct((N, C), jnp.float32),
    )(t, s, g, be, dist2d, degf2d, Wd, bd, Wdeg, bdeg, Wmh, Wmd, Wmg, bm)


def kernel(x, edge_index, edge_weight, dist_feat, degree_feat, W1, b1, g1,
           be1, W2, b2, g2, be2, Wd, bd, Wdeg, bdeg, Wm, bm):
    src = edge_index[0]
    dst = edge_index[1]

    dinv = _sc_degnorm(dst, edge_weight)
    dinv2d = dinv[:N].reshape(N, 1)

    h1p = _tc_linear_scale(x, W1, dinv2d)
    z1 = _sc_agg(h1p, src, dst, edge_weight)[:N]
    t1, s1 = _tc_stats(z1, h1p, dinv2d, b1.reshape(1, C))
    h2p = _tc_bn_relu_linear_scale(t1, s1, g1.reshape(1, C), be1.reshape(1, C),
                                   W2, dinv2d)
    z2 = _sc_agg(h2p, src, dst, edge_weight)[:N]
    t2, s2 = _tc_stats(z2, h2p, dinv2d, b2.reshape(1, C))
    out = _tc_final(t2, s2, g2.reshape(1, C), be2.reshape(1, C),
                    dist_feat.reshape(N, 1), degree_feat.reshape(N, 1),
                    Wd, bd.reshape(1, C), Wdeg, bdeg.reshape(1, C),
                    Wm[0:C], Wm[C:2 * C], Wm[2 * C:3 * C], bm.reshape(1, C))
    return out


# speedup vs baseline: 7.7661x; 7.7661x over previous
import dataclasses
import functools

import jax
import jax.numpy as jnp
from jax import lax
from jax.experimental import pallas as pl
from jax.experimental.pallas import tpu as pltpu
from jax.experimental.pallas import tpu_sc as plsc

N = 10000
E = 160000
IN_C = 256
C = 512
NPAD = 10240
NSUB = 16
NCORE = 2
NPASS = 2
LANES = 16
EPT = E // NSUB
ROWS_PER_TILE = NPAD // (NSUB * NCORE * NPASS)
KB = 16
NBUF = 4
SEG = 2000
FLUSH_T = 512
CBUF = FLUSH_T + SEG + 2 * NBUF * KB
RB = 1000
GRID = N // RB

_mesh = plsc.VectorSubcoreMesh(core_axis_name="c", subcore_axis_name="s")

_sc_params = pltpu.CompilerParams()
if "needs_layout_passes" in pltpu.CompilerParams.__dataclass_fields__:
    _sc_params = dataclasses.replace(_sc_params, needs_layout_passes=False)


def _vec_loop(total):
    return range(total // LANES)


def _rsqrt_newton(x):
    i = plsc.bitcast(x, jnp.int32)
    i = jnp.int32(0x5F3759DF) - lax.shift_right_logical(i, 1)
    y = plsc.bitcast(i, jnp.float32)
    for _ in range(3):
        y = y * (1.5 - 0.5 * x * y * y)
    return y


def _sc_degnorm_body(dst_hbm, ew_hbm, dinv_hbm, dst_v, w_v, deg_v, col_v, acc_v,
                     stage_sh):
    cid = lax.axis_index("c")
    sid = lax.axis_index("s")

    @pl.when(cid == 0)
    def _():
        base = sid * EPT
        pltpu.sync_copy(dst_hbm.at[pl.ds(base, EPT)], dst_v)
        pltpu.sync_copy(ew_hbm.at[pl.ds(base, EPT)], w_v)
        zero = jnp.zeros((LANES,), jnp.float32)

        @pl.loop(0, NPAD // LANES)
        def _(j):
            deg_v[pl.ds(j * LANES, LANES)] = zero

        @pl.loop(0, EPT // LANES)
        def _(i):
            dvec = dst_v[pl.ds(i * LANES, LANES)]
            wvec = w_v[pl.ds(i * LANES, LANES)]
            plsc.addupdate_scatter(deg_v, [dvec], wvec)

        pltpu.sync_copy(deg_v, stage_sh.at[sid])
        plsc.subcore_barrier()

        width = NPAD // NSUB
        one = jnp.full((LANES,), 1.0, jnp.float32)
        for j in _vec_loop(width):
            acc_v[pl.ds(j * LANES, LANES)] = one
        for p in range(NSUB):
            pltpu.sync_copy(stage_sh.at[p].at[pl.ds(sid * width, width)], col_v)
            for j in _vec_loop(width):
                sl = pl.ds(j * LANES, LANES)
                acc_v[sl] = acc_v[sl] + col_v[sl]
        for j in _vec_loop(width):
            sl = pl.ds(j * LANES, LANES)
            acc_v[sl] = _rsqrt_newton(acc_v[sl])
        pltpu.sync_copy(acc_v, dinv_hbm.at[pl.ds(sid * width, width)])


def _sc_degnorm(dst, ew):
    k = pl.kernel(
        _sc_degnorm_body,
        out_type=jax.ShapeDtypeStruct((NPAD,), jnp.float32),
        mesh=_mesh,
        compiler_params=_sc_params,
        scratch_types=[
            pltpu.VMEM((EPT,), jnp.int32),
            pltpu.VMEM((EPT,), jnp.float32),
            pltpu.VMEM((NPAD,), jnp.float32),
            pltpu.VMEM((NPAD // NSUB,), jnp.float32),
            pltpu.VMEM((NPAD // NSUB,), jnp.float32),
            pltpu.VMEM_SHARED((NSUB, NPAD), jnp.float32),
        ],
    )
    return k(dst, ew)


def _sc_agg_body(hp_hbm, src_hbm, dst_hbm, ew_hbm, z_hbm, sseg_v, dseg_v,
                 wseg_v, csrc_v, cdst_v, cw_v, g0, g1, g2, g3, acc_v,
                 x0, x1, x2, x3, s0, s1, s2, s3):
    cid = lax.axis_index("c")
    sid = lax.axis_index("s")
    tid = cid * NSUB + sid
    zvec = jnp.zeros((LANES,), jnp.float32)
    zi = jnp.zeros((LANES,), jnp.int32)
    iota = lax.iota(jnp.int32, LANES)

    def accum(off, gbuf):
        @pl.loop(0, KB)
        def _(k):
            idx = lax.broadcast(off + k, (LANES,))
            base = plsc.load_gather(cdst_v, [idx]) * C + iota
            wb = plsc.load_gather(cw_v, [idx])
            vals = [wb * gbuf[k, pl.ds(j * LANES, LANES)]
                    for j in _vec_loop(C)]
            for j, v in enumerate(vals):
                plsc.addupdate_scatter(acc_v, [base + (j * LANES)], v)

    @pl.loop(0, NPASS)
    def _(p):
        g = p * (NSUB * NCORE) + tid
        r0 = g * ROWS_PER_TILE

        @pl.loop(0, (ROWS_PER_TILE * C) // LANES)
        def _(r):
            acc_v[pl.ds(r * LANES, LANES)] = zvec

        bufs = (g0, g1, g2, g3)
        idxs = (x0, x1, x2, x3)
        sems = (s0, s1, s2, s3)

        def stage_issue(off, sidx, gbuf, sem):
            sidx[0, pl.ds(0, KB)] = csrc_v[pl.ds(off, KB)]
            pltpu.async_copy(hp_hbm.at[sidx.at[0]], gbuf, sem)

        def flush(nh, nbF):
            nq = (nbF + (NBUF - 1)) // NBUF

            @pl.when(nbF > 0)
            def _():
                for b in range(NBUF):
                    @pl.when(b < nbF)
                    def _():
                        stage_issue(b * KB, idxs[b], bufs[b], sems[b])

            def ring(i, _):
                for b in range(NBUF):
                    bi = i * NBUF + b

                    @pl.when(bi < nbF)
                    def _():
                        pltpu.make_async_copy(hp_hbm.at[idxs[b].at[0]],
                                              bufs[b], sems[b]).wait()
                        accum(bi * KB, bufs[b])

                        @pl.when(bi + NBUF < nbF)
                        def _():
                            stage_issue((bi + NBUF) * KB, idxs[b], bufs[b],
                                        sems[b])
                return 0

            lax.fori_loop(0, nq, ring, 0)
            moved = nbF * KB

            @pl.when(nbF > 0)
            def _():
                csrc_v[pl.ds(0, LANES)] = csrc_v[pl.ds(moved, LANES)]
                cdst_v[pl.ds(0, LANES)] = cdst_v[pl.ds(moved, LANES)]
                cw_v[pl.ds(0, LANES)] = cw_v[pl.ds(moved, LANES)]

            return nh - moved

        def segbody(seg, nh):
            soff = seg * SEG
            pltpu.async_copy(src_hbm.at[pl.ds(soff, SEG)], sseg_v, s0)
            pltpu.async_copy(dst_hbm.at[pl.ds(soff, SEG)], dseg_v, s0)
            pltpu.async_copy(ew_hbm.at[pl.ds(soff, SEG)], wseg_v, s0)
            pltpu.make_async_copy(src_hbm.at[pl.ds(soff, SEG)], sseg_v,
                                  s0).wait()
            pltpu.make_async_copy(dst_hbm.at[pl.ds(soff, SEG)], dseg_v,
                                  s0).wait()
            pltpu.make_async_copy(ew_hbm.at[pl.ds(soff, SEG)], wseg_v,
                                  s0).wait()

            def fbody(i, nh):
                sl = pl.ds(i * LANES, LANES)
                dvec = dseg_v[sl]
                m = (dvec >= r0) & (dvec < r0 + ROWS_PER_TILE)
                plsc.store_compressed(csrc_v.at[pl.ds(nh, LANES)], sseg_v[sl],
                                      mask=m)
                plsc.store_compressed(cdst_v.at[pl.ds(nh, LANES)], dvec - r0,
                                      mask=m)
                plsc.store_compressed(cw_v.at[pl.ds(nh, LANES)], wseg_v[sl],
                                      mask=m)
                return nh + lax.reduce_sum(m.astype(jnp.int32), axes=(0,))

            nh = lax.fori_loop(0, SEG // LANES, fbody, nh)
            nbF = jnp.where(nh >= FLUSH_T, nh // KB, 0)
            return flush(nh, nbF)

        nh = lax.fori_loop(0, E // SEG, segbody, jnp.int32(0))

        zi16 = jnp.zeros((LANES,), jnp.int32)
        for j in range((NBUF * KB) // LANES):
            sl = pl.ds(nh + j * LANES, LANES)
            csrc_v[sl] = zi16
            cdst_v[sl] = zi16
            cw_v[sl] = zvec
        flush(nh, (nh + (KB - 1)) // KB)

        pltpu.sync_copy(acc_v, z_hbm.at[pl.ds(r0 * C, ROWS_PER_TILE * C)])


def _sc_agg(hp, src, dst, ew):
    k = pl.kernel(
        _sc_agg_body,
        out_type=jax.ShapeDtypeStruct((NPAD * C,), jnp.float32),
        mesh=_mesh,
        compiler_params=_sc_params,
        scratch_types=[
            pltpu.VMEM((SEG,), jnp.int32),
            pltpu.VMEM((SEG,), jnp.int32),
            pltpu.VMEM((SEG,), jnp.float32),
            pltpu.VMEM((CBUF,), jnp.int32),
            pltpu.VMEM((CBUF,), jnp.int32),
            pltpu.VMEM((CBUF,), jnp.float32),
            pltpu.VMEM((KB, C), jnp.float32),
            pltpu.VMEM((KB, C), jnp.float32),
            pltpu.VMEM((KB, C), jnp.float32),
            pltpu.VMEM((KB, C), jnp.float32),
            pltpu.VMEM((ROWS_PER_TILE * C,), jnp.float32),
            pltpu.VMEM((1, KB), jnp.int32),
            pltpu.VMEM((1, KB), jnp.int32),
            pltpu.VMEM((1, KB), jnp.int32),
            pltpu.VMEM((1, KB), jnp.int32),
            pltpu.SemaphoreType.DMA,
            pltpu.SemaphoreType.DMA,
            pltpu.SemaphoreType.DMA,
            pltpu.SemaphoreType.DMA,
        ],
    )
    return k(hp, src, dst, ew).reshape(NPAD, C)


def _mm_scale_body(x_ref, w_ref, dinv_ref, o_ref):
    o_ref[...] = dinv_ref[...] * jnp.dot(
        x_ref[...], w_ref[...], preferred_element_type=jnp.float32)


def _tc_linear_scale(x, W, dinv2d):
    return pl.pallas_call(
        _mm_scale_body,
        grid=(GRID,),
        in_specs=[
            pl.BlockSpec((RB, x.shape[1]), lambda i: (i, 0)),
            pl.BlockSpec(W.shape, lambda i: (0, 0)),
            pl.BlockSpec((RB, 1), lambda i: (i, 0)),
        ],
        out_specs=pl.BlockSpec((RB, C), lambda i: (i, 0)),
        out_shape=jax.ShapeDtypeStruct((N, C), jnp.float32),
    )(x, W, dinv2d)


def _stats_body(z_ref, hp_ref, dinv_ref, b_ref, t_ref, s_ref):
    t = dinv_ref[...] * (z_ref[...] + hp_ref[...]) + b_ref[...]
    t_ref[...] = t
    part = jnp.concatenate(
        [jnp.sum(t, axis=0, keepdims=True),
         jnp.sum(t * t, axis=0, keepdims=True)], axis=0)

    @pl.when(pl.program_id(0) == 0)
    def _():
        s_ref[...] = part

    @pl.when(pl.program_id(0) != 0)
    def _():
        s_ref[...] = s_ref[...] + part


def _tc_stats(z, hp, dinv2d, b):
    return pl.pallas_call(
        _stats_body,
        grid=(GRID,),
        in_specs=[
            pl.BlockSpec((RB, C), lambda i: (i, 0)),
            pl.BlockSpec((RB, C), lambda i: (i, 0)),
            pl.BlockSpec((RB, 1), lambda i: (i, 0)),
            pl.BlockSpec((1, C), lambda i: (0, 0)),
        ],
        out_specs=[
            pl.BlockSpec((RB, C), lambda i: (i, 0)),
            pl.BlockSpec((2, C), lambda i: (0, 0)),
        ],
        out_shape=[
            jax.ShapeDtypeStruct((N, C), jnp.float32),
            jax.ShapeDtypeStruct((2, C), jnp.float32),
        ],
    )(z, hp, dinv2d, b)


def _bn_from_sums(t, s, g, be):
    m = s[0:1, :] * (1.0 / N)
    v = s[1:2, :] * (1.0 / N) - m * m
    return jnp.maximum((t - m) * lax.rsqrt(v + 1e-5) * g + be, 0.0)


def _bnmm_body(t_ref, s_ref, g_ref, be_ref, w_ref, dinv_ref, o_ref):
    u = _bn_from_sums(t_ref[...], s_ref[...], g_ref[...], be_ref[...])
    o_ref[...] = dinv_ref[...] * jnp.dot(
        u, w_ref[...], preferred_element_type=jnp.float32)


def _tc_bn_relu_linear_scale(t, s, g, be, W, dinv2d):
    return pl.pallas_call(
        _bnmm_body,
        grid=(GRID,),
        in_specs=[
            pl.BlockSpec((RB, C), lambda i: (i, 0)),
            pl.BlockSpec((2, C), lambda i: (0, 0)),
            pl.BlockSpec((1, C), lambda i: (0, 0)),
            pl.BlockSpec((1, C), lambda i: (0, 0)),
            pl.BlockSpec((C, C), lambda i: (0, 0)),
            pl.BlockSpec((RB, 1), lambda i: (i, 0)),
        ],
        out_specs=pl.BlockSpec((RB, C), lambda i: (i, 0)),
        out_shape=jax.ShapeDtypeStruct((N, C), jnp.float32),
    )(t, s, g, be, W, dinv2d)


def _final_body(t_ref, s_ref, g_ref, be_ref, dist_ref, degf_ref, wd_ref,
                bd_ref, wdeg_ref, bdeg_ref, wmh_ref, wmd_ref, wmg_ref, bm_ref,
                o_ref):
    u = _bn_from_sums(t_ref[...], s_ref[...], g_ref[...], be_ref[...])
    dfe = jnp.maximum(dist_ref[...] * wd_ref[...] + bd_ref[...], 0.0)
    gfe = jnp.maximum(degf_ref[...] * wdeg_ref[...] + bdeg_ref[...], 0.0)
    acc = jnp.dot(u, wmh_ref[...], preferred_element_type=jnp.float32)
    acc = acc + jnp.dot(dfe, wmd_ref[...], preferred_element_type=jnp.float32)
    acc = acc + jnp.dot(gfe, wmg_ref[...], preferred_element_type=jnp.float32)
    o_ref[...] = acc + bm_ref[...]


def _tc_final(t, s, g, be, dist2d, degf2d, Wd, bd, Wdeg, bdeg, Wmh, Wmd, Wmg,
              bm):
    col = pl.BlockSpec((RB, 1), lambda i: (i, 0))
    rowv = pl.BlockSpec((1, C), lambda i: (0, 0))
    mat = pl.BlockSpec((C, C), lambda i: (0, 0))
    return pl.pallas_call(
        _final_body,
        grid=(GRID,),
        in_specs=[
            pl.BlockSpec((RB, C), lambda i: (i, 0)),
            pl.BlockSpec((2, C), lambda i: (0, 0)),
            rowv, rowv, col, col, rowv, rowv, rowv, rowv, mat, mat, mat, rowv,
        ],
        out_specs=pl.BlockSpec((RB, C), lambda i: (i, 0)),
        out_shape=jax.ShapeDtypeStruct((N, C), jnp.float32),
    )(t, s, g, be, dist2d, degf2d, Wd, bd, Wdeg, bdeg, Wmh, Wmd, Wmg, bm)


def kernel(x, edge_index, edge_weight, dist_feat, degree_feat, W1, b1, g1,
           be1, W2, b2, g2, be2, Wd, bd, Wdeg, bdeg, Wm, bm):
    src = edge_index[0]
    dst = edge_index[1]

    dinv = _sc_degnorm(dst, edge_weight)
    dinv2d = dinv[:N].reshape(N, 1)

    h1p = _tc_linear_scale(x, W1, dinv2d)
    z1 = _sc_agg(h1p, src, dst, edge_weight)[:N]
    t1, s1 = _tc_stats(z1, h1p, dinv2d, b1.reshape(1, C))
    h2p = _tc_bn_relu_linear_scale(t1, s1, g1.reshape(1, C), be1.reshape(1, C),
                                   W2, dinv2d)
    z2 = _sc_agg(h2p, src, dst, edge_weight)[:N]
    t2, s2 = _tc_stats(z2, h2p, dinv2d, b2.reshape(1, C))
    out = _tc_final(t2, s2, g2.reshape(1, C), be2.reshape(1, C),
                    dist_feat.reshape(N, 1), degree_feat.reshape(N, 1),
                    Wd, bd.reshape(1, C), Wdeg, bdeg.reshape(1, C),
                    Wm[0:C], Wm[C:2 * C], Wm[2 * C:3 * C], bm.reshape(1, C))
    return out
